# Initial kernel scaffold; baseline (speedup 1.0000x reference)
import jax
import jax.numpy as jnp
from jax.experimental import pallas as pl


def kernel(x, z, h_index, r_index, edge_src, edge_type, edge_dst, fcvx_w1, fcvx_b1, fcvx_w2, fcvx_b2, l0_fcz_w, l0_fcz_b, l0_fco_w1, l0_fco_b1, l0_fco_w2, l0_fco_b2, l0_beta, l0_ln_g, l0_ln_b, l1_fcz_w, l1_fcz_b, l1_fco_w1, l1_fco_b1, l1_fco_w2, l1_fco_b2, l1_beta, l1_ln_g, l1_ln_b):
    raise NotImplementedError("write your pallas kernel here")


# speedup vs baseline: 3.7857x; 3.7857x over previous
import functools

import jax
import jax.numpy as jnp
from jax import lax
from jax.experimental import pallas as pl
from jax.experimental.pallas import tpu as pltpu
from jax.experimental.pallas import tpu_sc as plsc

F32 = jnp.float32


_BLK = 2000


def _embed_body(hidx, x_ref, w1_ref, b1_ref, corr_ref, w2_ref, b2_ref, o_ref):
    b = pl.program_id(0)
    i = pl.program_id(1)
    xb = x_ref[0]
    t = jnp.dot(xb, w1_ref[...], preferred_element_type=F32) + b1_ref[...]
    rows = lax.broadcasted_iota(jnp.int32, (xb.shape[0], 1), 0) + i * xb.shape[0]
    t = t + jnp.where(rows == hidx[b], 1.0, 0.0) * corr_ref[...]
    t = jnp.maximum(t, 0.0)
    o_ref[0] = jnp.dot(t, w2_ref[...], preferred_element_type=F32) + b2_ref[...]


def _embed(x, h_index, w1a, b1, corr, w2, b2):
    B, V, D = x.shape
    nb = V // _BLK
    grid = (B, nb)
    full = lambda a: pl.BlockSpec(a.shape, lambda b, i: (0,) * a.ndim)
    return pl.pallas_call(
        _embed_body,
        grid=grid,
        in_specs=[
            pl.BlockSpec(memory_space=pltpu.SMEM),
            pl.BlockSpec((1, _BLK, D), lambda b, i: (b, i, 0)),
            full(w1a), full(b1), full(corr), full(w2), full(b2),
        ],
        out_specs=pl.BlockSpec((1, _BLK, D), lambda b, i: (b, i, 0)),
        out_shape=jax.ShapeDtypeStruct((B, V, D), F32),
    )(h_index, x, w1a, b1, corr, w2, b2)


def _zr_body(z_ref, w0_ref, b0_ref, w1_ref, b1_ref, o0_ref, o1_ref):
    z = z_ref[...]
    o0_ref[...] = jnp.dot(z, w0_ref[...], preferred_element_type=F32) + b0_ref[...]
    o1_ref[...] = jnp.dot(z, w1_ref[...], preferred_element_type=F32) + b1_ref[...]


def _zr(z, w0, b0, w1, b1):
    B, D = z.shape
    RD = w0.shape[1]
    out = jax.ShapeDtypeStruct((B, RD), F32)
    return pl.pallas_call(_zr_body, out_shape=[out, out])(z, w0, b0, w1, b1)


def _ybuild_body(vx_ref, zr_ref, y_ref):
    r = pl.program_id(2)
    zr = zr_ref[0]
    mask = lax.broadcasted_iota(jnp.int32, (zr.shape[0], 1), 0) == r
    zrow = jnp.sum(jnp.where(mask, zr, 0.0), axis=0, keepdims=True)
    y_ref[0, 0] = zrow * vx_ref[0]


def _ybuild(vx, zr):
    B, V, D = vx.shape
    R = zr.shape[1]
    nb = V // _BLK
    return pl.pallas_call(
        _ybuild_body,
        grid=(B, nb, R),
        in_specs=[
            pl.BlockSpec((1, _BLK, D), lambda b, i, r: (b, i, 0)),
            pl.BlockSpec((1, R, D), lambda b, i, r: (b, 0, 0)),
        ],
        out_specs=pl.BlockSpec((1, 1, _BLK, D), lambda b, i, r: (b, r, i, 0)),
        out_shape=jax.ShapeDtypeStruct((B, R, V, D), F32),
    )(vx, zr)


def _post_body(agg_ref, vx_ref, w1_ref, b1_ref, w2_ref, b2_ref, beta_ref,
               g_ref, bb_ref, o_ref):
    vx = vx_ref[0]
    h = agg_ref[0] + beta_ref[...] * vx
    m = jnp.maximum(jnp.dot(h, w1_ref[...], preferred_element_type=F32)
                    + b1_ref[...], 0.0)
    h2 = jnp.dot(m, w2_ref[...], preferred_element_type=F32) + b2_ref[...]
    mu = jnp.mean(h2, axis=-1, keepdims=True)
    var = jnp.mean((h2 - mu) ** 2, axis=-1, keepdims=True)
    ln = (h2 - mu) / jnp.sqrt(var + 1e-5) * g_ref[...] + bb_ref[...]
    o_ref[0] = ln + vx


def _post(agg, vx, w1, b1, w2, b2, beta, g, bb):
    B, V, D = vx.shape
    nb = V // _BLK
    full = lambda a: pl.BlockSpec(a.shape, lambda b, i: (0,) * a.ndim)
    blk = pl.BlockSpec((1, _BLK, D), lambda b, i: (b, i, 0))
    return pl.pallas_call(
        _post_body,
        grid=(B, nb),
        in_specs=[blk, blk, full(w1), full(b1), full(w2), full(b2),
                  full(beta), full(g), full(bb)],
        out_specs=blk,
        out_shape=jax.ShapeDtypeStruct((B, V, D), F32),
    )(agg, vx, w1, b1, w2, b2, beta, g, bb)


_NC, _NS, _L = 2, 16, 16
_K = 80


@functools.cache
def _rspmm(B, V, D, R, E):
    ept = E // _NS
    nch = ept // _K
    assert ept * _NS == E and nch * _K == ept
    vpt = -(-V // (_NS * 8)) * 8
    vp = vpt * _NS
    mesh = plsc.VectorSubcoreMesh(core_axis_name="c", subcore_axis_name="s",
                                  num_cores=_NC, num_subcores=_NS)

    def body(y_hbm, src_hbm, typ_hbm, dst_hbm, zero_hbm, out_hbm,
             acc, src_v, typ_v, comb_v, dst_v, rows_v, sem):
        c = lax.axis_index("c")
        s = lax.axis_index("s")
        boff = c * (R * V)
        pltpu.sync_copy(zero_hbm.at[pl.ds(s * vpt, vpt)],
                        acc.at[pl.ds(s * vpt, vpt)])
        plsc.subcore_barrier()

        base0 = s * ept

        @pl.loop(0, nch)
        def _chunk(i):
            base = base0 + i * _K
            pltpu.sync_copy(src_hbm.at[pl.ds(base, _K)], src_v)
            pltpu.sync_copy(typ_hbm.at[pl.ds(base, _K)], typ_v)
            pltpu.sync_copy(dst_hbm.at[pl.ds(base, _K)], dst_v)
            for j in range(_K // _L):
                sl = pl.ds(j * _L, _L)
                comb_v[sl] = typ_v[sl] * V + src_v[sl] + boff
            pltpu.async_copy(y_hbm.at[comb_v], rows_v, sem).wait()
            pltpu.sync_copy(rows_v, acc.at[dst_v], add=True)

        plsc.subcore_barrier()
        pltpu.sync_copy(acc.at[pl.ds(s * vpt, vpt)],
                        out_hbm.at[pl.ds(c * vp + s * vpt, vpt)])

    return pl.kernel(
        body,
        out_type=jax.ShapeDtypeStruct((B * vp, D), F32),
        mesh=mesh,
        scratch_types=[
            pltpu.VMEM_SHARED((vp, D), F32),
            pltpu.VMEM((_K,), jnp.int32),
            pltpu.VMEM((_K,), jnp.int32),
            pltpu.VMEM((_K,), jnp.int32),
            pltpu.VMEM((_K,), jnp.int32),
            pltpu.VMEM((_K, D), F32),
            pltpu.SemaphoreType.DMA,
        ],
    )


def kernel(x, z, h_index, r_index, edge_src, edge_type, edge_dst,
           fcvx_w1, fcvx_b1, fcvx_w2, fcvx_b2,
           l0_fcz_w, l0_fcz_b, l0_fco_w1, l0_fco_b1, l0_fco_w2, l0_fco_b2,
           l0_beta, l0_ln_g, l0_ln_b,
           l1_fcz_w, l1_fcz_b, l1_fco_w1, l1_fco_b1, l1_fco_w2, l1_fco_b2,
           l1_beta, l1_ln_g, l1_ln_b):
    B, V, D = x.shape
    R = l0_fcz_w.shape[1] // D
    E = edge_src.shape[0]

    row = lambda a: a.reshape(1, -1)
    corr = row(jnp.sum(fcvx_w1[D:], axis=0))
    vx = _embed(x, h_index.astype(jnp.int32), fcvx_w1[:D], row(fcvx_b1),
                corr, fcvx_w2, row(fcvx_b2))

    zr0, zr1 = _zr(z, l0_fcz_w, row(l0_fcz_b), l1_fcz_w, row(l1_fcz_b))
    zr0 = zr0.reshape(B, R, D)
    zr1 = zr1.reshape(B, R, D)

    rspmm = _rspmm(B, V, D, R, E)
    vp = -(-V // (_NS * 8)) * 8 * _NS
    zero = jnp.zeros((vp, D), F32)
    esrc = edge_src.astype(jnp.int32)
    etyp = edge_type.astype(jnp.int32)
    edst = edge_dst.astype(jnp.int32)

    for zr, w1, b1, w2, b2, beta, g, bb in (
        (zr0, l0_fco_w1, l0_fco_b1, l0_fco_w2, l0_fco_b2, l0_beta, l0_ln_g,
         l0_ln_b),
        (zr1, l1_fco_w1, l1_fco_b1, l1_fco_w2, l1_fco_b2, l1_beta, l1_ln_g,
         l1_ln_b),
    ):
        y = _ybuild(vx, zr).reshape(B * R * V, D)
        agg = rspmm(y, esrc, etyp, edst, zero).reshape(B, vp, D)[:, :V]
        vx = _post(agg, vx, w1, row(b1), w2, row(b2), beta, row(g), row(bb))
    return vx


# speedup vs baseline: 7.2670x; 1.9196x over previous
import functools

import jax
import jax.numpy as jnp
from jax import lax
from jax.experimental import pallas as pl
from jax.experimental.pallas import tpu as pltpu
from jax.experimental.pallas import tpu_sc as plsc

F32 = jnp.float32


_BLK = 2000


def _embed_body(hidx, x_ref, w1_ref, b1_ref, corr_ref, w2_ref, b2_ref, o_ref):
    b = pl.program_id(0)
    i = pl.program_id(1)
    xb = x_ref[0]
    t = jnp.dot(xb, w1_ref[...], preferred_element_type=F32) + b1_ref[...]
    rows = lax.broadcasted_iota(jnp.int32, (xb.shape[0], 1), 0) + i * xb.shape[0]
    t = t + jnp.where(rows == hidx[b], 1.0, 0.0) * corr_ref[...]
    t = jnp.maximum(t, 0.0)
    o_ref[0] = jnp.dot(t, w2_ref[...], preferred_element_type=F32) + b2_ref[...]


def _embed(x, h_index, w1a, b1, corr, w2, b2):
    B, V, D = x.shape
    nb = V // _BLK
    grid = (B, nb)
    full = lambda a: pl.BlockSpec(a.shape, lambda b, i: (0,) * a.ndim)
    return pl.pallas_call(
        _embed_body,
        grid=grid,
        in_specs=[
            pl.BlockSpec(memory_space=pltpu.SMEM),
            pl.BlockSpec((1, _BLK, D), lambda b, i: (b, i, 0)),
            full(w1a), full(b1), full(corr), full(w2), full(b2),
        ],
        out_specs=pl.BlockSpec((1, _BLK, D), lambda b, i: (b, i, 0)),
        out_shape=jax.ShapeDtypeStruct((B, V, D), F32),
    )(h_index, x, w1a, b1, corr, w2, b2)


def _zr_body(z_ref, w0_ref, b0_ref, w1_ref, b1_ref, o0_ref, o1_ref):
    z = z_ref[...]
    o0_ref[...] = jnp.dot(z, w0_ref[...], preferred_element_type=F32) + b0_ref[...]
    o1_ref[...] = jnp.dot(z, w1_ref[...], preferred_element_type=F32) + b1_ref[...]


def _zr(z, w0, b0, w1, b1):
    B, D = z.shape
    RD = w0.shape[1]
    out = jax.ShapeDtypeStruct((B, RD), F32)
    return pl.pallas_call(_zr_body, out_shape=[out, out])(z, w0, b0, w1, b1)


def _ybuild_body(vx_ref, zr_ref, y_ref):
    r = pl.program_id(2)
    zr = zr_ref[0]
    mask = lax.broadcasted_iota(jnp.int32, (zr.shape[0], 1), 0) == r
    zrow = jnp.sum(jnp.where(mask, zr, 0.0), axis=0, keepdims=True)
    y_ref[0, 0] = zrow * vx_ref[0]


def _ybuild(vx, zr):
    B, V, D = vx.shape
    R = zr.shape[1]
    nb = V // _BLK
    return pl.pallas_call(
        _ybuild_body,
        grid=(B, nb, R),
        in_specs=[
            pl.BlockSpec((1, _BLK, D), lambda b, i, r: (b, i, 0)),
            pl.BlockSpec((1, R, D), lambda b, i, r: (b, 0, 0)),
        ],
        out_specs=pl.BlockSpec((1, 1, _BLK, D), lambda b, i, r: (b, r, i, 0)),
        out_shape=jax.ShapeDtypeStruct((B, R, V, D), F32),
    )(vx, zr)


def _post_body(agg_ref, vx_ref, w1_ref, b1_ref, w2_ref, b2_ref, beta_ref,
               g_ref, bb_ref, o_ref):
    vx = vx_ref[0]
    h = agg_ref[0] + beta_ref[...] * vx
    m = jnp.maximum(jnp.dot(h, w1_ref[...], preferred_element_type=F32)
                    + b1_ref[...], 0.0)
    h2 = jnp.dot(m, w2_ref[...], preferred_element_type=F32) + b2_ref[...]
    mu = jnp.mean(h2, axis=-1, keepdims=True)
    var = jnp.mean((h2 - mu) ** 2, axis=-1, keepdims=True)
    ln = (h2 - mu) / jnp.sqrt(var + 1e-5) * g_ref[...] + bb_ref[...]
    o_ref[0] = ln + vx


def _post(agg, vx, w1, b1, w2, b2, beta, g, bb):
    B, V, D = vx.shape
    nb = V // _BLK
    full = lambda a: pl.BlockSpec(a.shape, lambda b, i: (0,) * a.ndim)
    blk = pl.BlockSpec((1, _BLK, D), lambda b, i: (b, i, 0))
    return pl.pallas_call(
        _post_body,
        grid=(B, nb),
        in_specs=[blk, blk, full(w1), full(b1), full(w2), full(b2),
                  full(beta), full(g), full(bb)],
        out_specs=blk,
        out_shape=jax.ShapeDtypeStruct((B, V, D), F32),
    )(agg, vx, w1, b1, w2, b2, beta, g, bb)


_NC, _NS, _L = 2, 16, 16
_K = 80


@functools.cache
def _rspmm(B, V, D, R, E):
    ept = E // _NS
    nch = ept // _K
    assert ept * _NS == E and nch * _K == ept
    vpt = -(-V // (_NS * 8)) * 8
    vp = vpt * _NS
    mesh = plsc.VectorSubcoreMesh(core_axis_name="c", subcore_axis_name="s",
                                  num_cores=_NC, num_subcores=_NS)

    npairs = nch // 2
    assert npairs * 2 == nch

    def body(y_hbm, idx_hbm, zero_hbm, out_hbm,
             acc, idx0, idx1, comb0, comb1, rows0, rows1,
             semx0, semx1, sem0, sem1):
        c = lax.axis_index("c")
        s = lax.axis_index("s")
        boff = c * (R * V)
        ch0 = s * nch

        def idx_load(i, buf, sem):
            pltpu.async_copy(idx_hbm.at[ch0 + i], buf, sem)

        def idx_wait(buf, sem):
            pltpu.make_async_copy(idx_hbm.at[0], buf, sem).wait()

        def stage(buf, comb_v):
            for j in range(_K // _L):
                sl = pl.ds(j * _L, _L)
                comb_v[sl] = buf[1, sl] * V + buf[0, sl] + boff

        def gather(comb_v, rows_v, sem):
            pltpu.async_copy(y_hbm.at[comb_v], rows_v, sem)

        def gather_wait(comb_v, rows_v, sem):
            pltpu.make_async_copy(y_hbm.at[comb_v], rows_v, sem).wait()

        def scatter(rows_v, buf):
            pltpu.sync_copy(rows_v, acc.at[buf.at[2]], add=True)

        idx_load(0, idx0, semx0)
        idx_load(1, idx1, semx1)
        pltpu.sync_copy(zero_hbm.at[pl.ds(s * vpt, vpt)],
                        acc.at[pl.ds(s * vpt, vpt)])
        plsc.subcore_barrier()
        idx_wait(idx0, semx0)
        stage(idx0, comb0)
        gather(comb0, rows0, sem0)

        @pl.loop(0, npairs)
        def _pair(h):
            i0 = h * 2
            last = h >= npairs - 1
            idx_wait(idx1, semx1)
            stage(idx1, comb1)
            gather_wait(comb0, rows0, sem0)
            gather(comb1, rows1, sem1)
            scatter(rows0, idx0)

            @pl.when(jnp.logical_not(last))
            def _():
                idx_load(i0 + 2, idx0, semx0)

            @pl.when(jnp.logical_not(last))
            def _():
                idx_wait(idx0, semx0)
                stage(idx0, comb0)
            gather_wait(comb1, rows1, sem1)

            @pl.when(jnp.logical_not(last))
            def _():
                gather(comb0, rows0, sem0)
            scatter(rows1, idx1)

            @pl.when(jnp.logical_not(last))
            def _():
                idx_load(i0 + 3, idx1, semx1)

        plsc.subcore_barrier()
        pltpu.sync_copy(acc.at[pl.ds(s * vpt, vpt)],
                        out_hbm.at[pl.ds(c * vp + s * vpt, vpt)])

    return pl.kernel(
        body,
        out_type=jax.ShapeDtypeStruct((B * vp, D), F32),
        mesh=mesh,
        scratch_types=[
            pltpu.VMEM_SHARED((vp, D), F32),
            pltpu.VMEM((3, _K), jnp.int32),
            pltpu.VMEM((3, _K), jnp.int32),
            pltpu.VMEM((_K,), jnp.int32),
            pltpu.VMEM((_K,), jnp.int32),
            pltpu.VMEM((_K, D), F32),
            pltpu.VMEM((_K, D), F32),
            pltpu.SemaphoreType.DMA,
            pltpu.SemaphoreType.DMA,
            pltpu.SemaphoreType.DMA,
            pltpu.SemaphoreType.DMA,
        ],
    )


def kernel(x, z, h_index, r_index, edge_src, edge_type, edge_dst,
           fcvx_w1, fcvx_b1, fcvx_w2, fcvx_b2,
           l0_fcz_w, l0_fcz_b, l0_fco_w1, l0_fco_b1, l0_fco_w2, l0_fco_b2,
           l0_beta, l0_ln_g, l0_ln_b,
           l1_fcz_w, l1_fcz_b, l1_fco_w1, l1_fco_b1, l1_fco_w2, l1_fco_b2,
           l1_beta, l1_ln_g, l1_ln_b):
    B, V, D = x.shape
    R = l0_fcz_w.shape[1] // D
    E = edge_src.shape[0]

    row = lambda a: a.reshape(1, -1)
    corr = row(jnp.sum(fcvx_w1[D:], axis=0))
    vx = _embed(x, h_index.astype(jnp.int32), fcvx_w1[:D], row(fcvx_b1),
                corr, fcvx_w2, row(fcvx_b2))

    zr0, zr1 = _zr(z, l0_fcz_w, row(l0_fcz_b), l1_fcz_w, row(l1_fcz_b))
    zr0 = zr0.reshape(B, R, D)
    zr1 = zr1.reshape(B, R, D)

    rspmm = _rspmm(B, V, D, R, E)
    vp = -(-V // (_NS * 8)) * 8 * _NS
    zero = jnp.zeros((vp, D), F32)
    packed = jnp.stack(
        [edge_src.astype(jnp.int32).reshape(-1, _K),
         edge_type.astype(jnp.int32).reshape(-1, _K),
         edge_dst.astype(jnp.int32).reshape(-1, _K)], axis=1)

    for zr, w1, b1, w2, b2, beta, g, bb in (
        (zr0, l0_fco_w1, l0_fco_b1, l0_fco_w2, l0_fco_b2, l0_beta, l0_ln_g,
         l0_ln_b),
        (zr1, l1_fco_w1, l1_fco_b1, l1_fco_w2, l1_fco_b2, l1_beta, l1_ln_g,
         l1_ln_b),
    ):
        y = _ybuild(vx, zr).reshape(B * R * V, D)
        agg = rspmm(y, packed, zero).reshape(B, vp, D)[:, :V]
        vx = _post(agg, vx, w1, row(b1), w2, row(b2), beta, row(g), row(bb))
    return vx
